# baseline (speedup 1.0000x reference)
import functools

import jax
import jax.numpy as jnp
from jax import lax
from jax.experimental import pallas as pl
from jax.experimental.pallas import tpu as pltpu
from jax.experimental.pallas import tpu_sc as plsc

_DIM = 256
_N_EMBED = 8192
_N_TOKENS = 8192
_TM = 1024
_TN = 1024
_TB = 1024
_NBATCH = 8


def _argmin_body(x2_ref, e2_ref, xb_ref, em_ref, out_ref, rmin_ref, ridx_ref):
    j = pl.program_id(1)
    ncode = pl.num_programs(1)
    conv = lax.dot_general(xb_ref[...], em_ref[...], (((1,), (0,)), ((), ())),
                           preferred_element_type=jnp.float32)
    dist = (x2_ref[...] - conv) + e2_ref[...]
    tmin = jnp.min(dist, axis=1, keepdims=True)
    lane = lax.broadcasted_iota(jnp.int32, dist.shape, 1)
    cand = jnp.where(dist == tmin, lane, jnp.int32(2**30))
    targ = jnp.min(cand, axis=1, keepdims=True) + j * _TN

    @pl.when(j == 0)
    def _():
        rmin_ref[...] = tmin
        ridx_ref[...] = targ

    @pl.when(j > 0)
    def _():
        upd = tmin < rmin_ref[...]
        ridx_ref[...] = jnp.where(upd, targ, ridx_ref[...])
        rmin_ref[...] = jnp.where(upd, tmin, rmin_ref[...])

    @pl.when(j == ncode - 1)
    def _():
        out_ref[...] = ridx_ref[...]


def _argmin_call(x2, e2, xb, embed):
    return pl.pallas_call(
        _argmin_body,
        grid=(_N_TOKENS // _TM, _N_EMBED // _TN),
        in_specs=[
            pl.BlockSpec((_TM, 1), lambda i, j: (i, 0)),
            pl.BlockSpec((1, _TN), lambda i, j: (0, j)),
            pl.BlockSpec((_TM, _DIM), lambda i, j: (i, 0)),
            pl.BlockSpec((_DIM, _TN), lambda i, j: (0, j)),
        ],
        out_specs=pl.BlockSpec((_TM, 1), lambda i, j: (i, 0)),
        out_shape=jax.ShapeDtypeStruct((_N_TOKENS, 1), jnp.int32),
        scratch_shapes=[
            pltpu.VMEM((_TM, 1), jnp.float32),
            pltpu.VMEM((_TM, 1), jnp.int32),
        ],
        compiler_params=pltpu.CompilerParams(
            dimension_semantics=("arbitrary", "arbitrary")),
    )(x2, e2, xb, embed)


def _sc_gather(table, idx):
    info = plsc.get_sparse_core_info()
    ncores, nsub = info.num_cores, info.num_subcores
    nw = ncores * nsub
    bpw = _N_TOKENS // nw
    mesh = plsc.VectorSubcoreMesh(core_axis_name="c", subcore_axis_name="s")

    @functools.partial(
        pl.kernel, mesh=mesh,
        out_type=jax.ShapeDtypeStruct((_N_TOKENS, _DIM), jnp.float32),
        scratch_types=[
            pltpu.VMEM((bpw,), jnp.int32),
            pltpu.VMEM((bpw, _DIM), jnp.float32),
            pltpu.SemaphoreType.DMA,
        ],
    )
    def k(table_hbm, idx_hbm, out_hbm, idx_v, rows_v, sem):
        wid = lax.axis_index("s") * ncores + lax.axis_index("c")
        base = wid * bpw
        pltpu.sync_copy(idx_hbm.at[pl.ds(base, bpw)], idx_v)
        pltpu.async_copy(table_hbm.at[idx_v], rows_v, sem).wait()
        pltpu.sync_copy(rows_v, out_hbm.at[pl.ds(base, bpw)])

    return k(table, idx)


def _stats_body(ind_ref, fl_ref, q_ref, diff_ref, perp_ref, h_ref):
    b = pl.program_id(0)
    nb = pl.num_programs(0)

    d = q_ref[...] - fl_ref[...]
    val = jnp.sum(d * d) * (1.0 / (_TB * _DIM))
    lane8 = lax.broadcasted_iota(jnp.int32, (1, _NBATCH), 1)
    diff_ref[...] = jnp.where(lane8 == b, val, diff_ref[...])

    code = lax.broadcasted_iota(jnp.int32, (1, _TN), 1) + b * _TN

    def tok_step(t, acc):
        sub = ind_ref[pl.ds(t * _TM, _TM), :]
        eq = (sub == code).astype(jnp.float32)
        return acc + jnp.sum(eq, axis=0, keepdims=True)

    counts = lax.fori_loop(0, _N_TOKENS // _TM, tok_step,
                           jnp.zeros((1, _TN), jnp.float32))
    p = counts * (1.0 / _N_TOKENS)
    h_part = jnp.sum(p * jnp.log(p + 1e-10))
    h_prev = jnp.where(b == 0, 0.0, h_ref[0])
    h_new = h_prev + h_part
    h_ref[0] = h_new

    @pl.when(b == nb - 1)
    def _():
        perp_ref[...] = jnp.full((1, 1), jnp.exp(-h_new), jnp.float32)


def _stats_call(ind2, flatten, quantize):
    return pl.pallas_call(
        _stats_body,
        grid=(_NBATCH,),
        in_specs=[
            pl.BlockSpec((_N_TOKENS, 1), lambda b: (0, 0)),
            pl.BlockSpec((_TB, _DIM), lambda b: (b, 0)),
            pl.BlockSpec((_TB, _DIM), lambda b: (b, 0)),
        ],
        out_specs=[
            pl.BlockSpec((1, _NBATCH), lambda b: (0, 0)),
            pl.BlockSpec((1, 1), lambda b: (0, 0)),
        ],
        out_shape=[
            jax.ShapeDtypeStruct((1, _NBATCH), jnp.float32),
            jax.ShapeDtypeStruct((1, 1), jnp.float32),
        ],
        scratch_shapes=[pltpu.SMEM((1,), jnp.float32)],
        compiler_params=pltpu.CompilerParams(
            dimension_semantics=("arbitrary",)),
    )(ind2, flatten, quantize)


def kernel(x, embed):
    xp = jnp.transpose(x, (0, 2, 3, 1))
    flatten = xp.reshape(-1, _DIM)
    x2 = (flatten ** 2).sum(axis=1, keepdims=True)
    e2 = (embed ** 2).sum(axis=0, keepdims=True)
    xb = (2.0 * flatten).astype(jnp.bfloat16)
    ind2 = _argmin_call(x2, e2, xb, embed)
    quantize = _sc_gather(embed.T, ind2.reshape(_N_TOKENS))
    diff_row, perp11 = _stats_call(ind2, flatten, quantize)
    out0 = jnp.transpose(quantize.reshape(8, 32, 32, _DIM), (0, 3, 1, 2))
    return out0, diff_row.reshape(_NBATCH, 1), perp11.reshape(())


# speedup vs baseline: 1.0749x; 1.0749x over previous
import functools

import jax
import jax.numpy as jnp
from jax import lax
from jax.experimental import pallas as pl
from jax.experimental.pallas import tpu as pltpu
from jax.experimental.pallas import tpu_sc as plsc

_DIM = 256
_N_EMBED = 8192
_N_TOKENS = 8192
_TM = 1024
_TN = 1024
_TB = 1024
_NBATCH = 8


def _argmin_body(x2_ref, e2_ref, xb_ref, em_ref, out_ref, rmin_ref, ridx_ref):
    j = pl.program_id(1)
    ncode = pl.num_programs(1)
    conv = lax.dot_general(xb_ref[...], em_ref[...], (((1,), (0,)), ((), ())),
                           preferred_element_type=jnp.float32)
    dist = (x2_ref[...] - conv) + e2_ref[...]
    tmin = jnp.min(dist, axis=1, keepdims=True)
    lane = lax.broadcasted_iota(jnp.int32, dist.shape, 1)
    cand = jnp.where(dist == tmin, lane, jnp.int32(2**30))
    targ = jnp.min(cand, axis=1, keepdims=True) + j * _TN

    @pl.when(j == 0)
    def _():
        rmin_ref[...] = tmin
        ridx_ref[...] = targ

    @pl.when(j > 0)
    def _():
        upd = tmin < rmin_ref[...]
        ridx_ref[...] = jnp.where(upd, targ, ridx_ref[...])
        rmin_ref[...] = jnp.where(upd, tmin, rmin_ref[...])

    @pl.when(j == ncode - 1)
    def _():
        out_ref[...] = ridx_ref[...]


def _argmin_call(x2, e2, xb, embed):
    return pl.pallas_call(
        _argmin_body,
        grid=(_N_TOKENS // _TM, _N_EMBED // _TN),
        in_specs=[
            pl.BlockSpec((_TM, 1), lambda i, j: (i, 0)),
            pl.BlockSpec((1, _TN), lambda i, j: (0, j)),
            pl.BlockSpec((_TM, _DIM), lambda i, j: (i, 0)),
            pl.BlockSpec((_DIM, _TN), lambda i, j: (0, j)),
        ],
        out_specs=pl.BlockSpec((_TM, 1), lambda i, j: (i, 0)),
        out_shape=jax.ShapeDtypeStruct((_N_TOKENS, 1), jnp.int32),
        scratch_shapes=[
            pltpu.VMEM((_TM, 1), jnp.float32),
            pltpu.VMEM((_TM, 1), jnp.int32),
        ],
        compiler_params=pltpu.CompilerParams(
            dimension_semantics=("arbitrary", "arbitrary")),
    )(x2, e2, xb, embed)


def _sc_gather(table, idx):
    info = plsc.get_sparse_core_info()
    ncores, nsub = info.num_cores, info.num_subcores
    nw = ncores * nsub
    bpw = _N_TOKENS // nw
    mesh = plsc.VectorSubcoreMesh(core_axis_name="c", subcore_axis_name="s")

    @functools.partial(
        pl.kernel, mesh=mesh,
        out_type=jax.ShapeDtypeStruct((_N_TOKENS, _DIM), jnp.float32),
        scratch_types=[
            pltpu.VMEM((bpw,), jnp.int32),
            pltpu.VMEM((bpw, _DIM), jnp.float32),
            pltpu.SemaphoreType.DMA,
        ],
    )
    def k(table_hbm, idx_hbm, out_hbm, idx_v, rows_v, sem):
        wid = lax.axis_index("s") * ncores + lax.axis_index("c")
        base = wid * bpw
        pltpu.sync_copy(idx_hbm.at[pl.ds(base, bpw)], idx_v)
        pltpu.async_copy(table_hbm.at[idx_v], rows_v, sem).wait()
        pltpu.sync_copy(rows_v, out_hbm.at[pl.ds(base, bpw)])

    return k(table, idx)


def _hist_body(ind_ref, perp_ref, h_ref):
    b = pl.program_id(0)
    nb = pl.num_programs(0)
    code = lax.broadcasted_iota(jnp.int32, (1, _TN), 1) + b * _TN

    def tok_step(t, acc):
        sub = ind_ref[pl.ds(t * _TM, _TM), :]
        eq = (sub == code).astype(jnp.float32)
        return acc + jnp.sum(eq, axis=0, keepdims=True)

    counts = lax.fori_loop(0, _N_TOKENS // _TM, tok_step,
                           jnp.zeros((1, _TN), jnp.float32))
    p = counts * (1.0 / _N_TOKENS)
    h_part = jnp.sum(p * jnp.log(p + 1e-10))
    h_prev = jnp.where(b == 0, 0.0, h_ref[0])
    h_new = h_prev + h_part
    h_ref[0] = h_new

    @pl.when(b == nb - 1)
    def _():
        perp_ref[...] = jnp.full((1, 1), jnp.exp(-h_new), jnp.float32)


def _hist_call(ind2):
    return pl.pallas_call(
        _hist_body,
        grid=(_NBATCH,),
        in_specs=[pl.BlockSpec((_N_TOKENS, 1), lambda b: (0, 0))],
        out_specs=pl.BlockSpec((1, 1), lambda b: (0, 0)),
        out_shape=jax.ShapeDtypeStruct((1, 1), jnp.float32),
        scratch_shapes=[pltpu.SMEM((1,), jnp.float32)],
        compiler_params=pltpu.CompilerParams(
            dimension_semantics=("arbitrary",)),
    )(ind2)


def _diff_body(fl_ref, q_ref, diff_ref):
    b = pl.program_id(0)
    d = q_ref[...] - fl_ref[...]
    val = jnp.sum(d * d) * (1.0 / (_TB * _DIM))
    lane8 = lax.broadcasted_iota(jnp.int32, (1, _NBATCH), 1)
    diff_ref[...] = jnp.where(lane8 == b, val, diff_ref[...])


def _diff_call(flatten, quantize):
    return pl.pallas_call(
        _diff_body,
        grid=(_NBATCH,),
        in_specs=[
            pl.BlockSpec((_TB, _DIM), lambda b: (b, 0)),
            pl.BlockSpec((_TB, _DIM), lambda b: (b, 0)),
        ],
        out_specs=pl.BlockSpec((1, _NBATCH), lambda b: (0, 0)),
        out_shape=jax.ShapeDtypeStruct((1, _NBATCH), jnp.float32),
        compiler_params=pltpu.CompilerParams(
            dimension_semantics=("arbitrary",)),
    )(flatten, quantize)


def kernel(x, embed):
    xp = jnp.transpose(x, (0, 2, 3, 1))
    flatten = xp.reshape(-1, _DIM)
    x2 = (flatten ** 2).sum(axis=1, keepdims=True)
    e2 = (embed ** 2).sum(axis=0, keepdims=True)
    xb = (2.0 * flatten).astype(jnp.bfloat16)
    ind2 = _argmin_call(x2, e2, xb, embed)
    perp11 = _hist_call(ind2)
    quantize = _sc_gather(embed.T, ind2.reshape(_N_TOKENS))
    diff_row = _diff_call(flatten, quantize)
    out0 = jnp.transpose(quantize.reshape(8, 32, 32, _DIM), (0, 3, 1, 2))
    return out0, diff_row.reshape(_NBATCH, 1), perp11.reshape(())


# speedup vs baseline: 1.1425x; 1.0629x over previous
import functools

import jax
import jax.numpy as jnp
from jax import lax
from jax.experimental import pallas as pl
from jax.experimental.pallas import tpu as pltpu
from jax.experimental.pallas import tpu_sc as plsc

_DIM = 256
_N_EMBED = 8192
_N_TOKENS = 8192
_TM = 1024
_TN = 1024
_TB = 1024
_NBATCH = 8


def _argmin_body(e2_ref, xb_ref, em_ref, out_ref, rmin_ref, ridx_ref):
    j = pl.program_id(1)
    ncode = pl.num_programs(1)
    conv = lax.dot_general(xb_ref[...], em_ref[...], (((1,), (0,)), ((), ())),
                           preferred_element_type=jnp.float32)
    dist = e2_ref[...] - conv
    tmin = jnp.min(dist, axis=1, keepdims=True)
    lane = lax.broadcasted_iota(jnp.int32, dist.shape, 1)
    cand = jnp.where(dist == tmin, lane, jnp.int32(2**30))
    targ = jnp.min(cand, axis=1, keepdims=True) + j * _TN

    @pl.when(j == 0)
    def _():
        rmin_ref[...] = tmin
        ridx_ref[...] = targ

    @pl.when(j > 0)
    def _():
        upd = tmin < rmin_ref[...]
        ridx_ref[...] = jnp.where(upd, targ, ridx_ref[...])
        rmin_ref[...] = jnp.where(upd, tmin, rmin_ref[...])

    @pl.when(j == ncode - 1)
    def _():
        out_ref[...] = ridx_ref[...]


def _argmin_call(e2, xb, embed):
    return pl.pallas_call(
        _argmin_body,
        grid=(_N_TOKENS // _TM, _N_EMBED // _TN),
        in_specs=[
            pl.BlockSpec((1, _TN), lambda i, j: (0, j)),
            pl.BlockSpec((_TM, _DIM), lambda i, j: (i, 0)),
            pl.BlockSpec((_DIM, _TN), lambda i, j: (0, j)),
        ],
        out_specs=pl.BlockSpec((_TM, 1), lambda i, j: (i, 0)),
        out_shape=jax.ShapeDtypeStruct((_N_TOKENS, 1), jnp.int32),
        scratch_shapes=[
            pltpu.VMEM((_TM, 1), jnp.float32),
            pltpu.VMEM((_TM, 1), jnp.int32),
        ],
        compiler_params=pltpu.CompilerParams(
            dimension_semantics=("arbitrary", "arbitrary")),
    )(e2, xb, embed)


def _sc_gather(table, idx):
    info = plsc.get_sparse_core_info()
    ncores, nsub = info.num_cores, info.num_subcores
    nw = ncores * nsub
    bpw = _N_TOKENS // nw
    mesh = plsc.VectorSubcoreMesh(core_axis_name="c", subcore_axis_name="s")

    @functools.partial(
        pl.kernel, mesh=mesh,
        out_type=jax.ShapeDtypeStruct((_N_TOKENS, _DIM), jnp.float32),
        scratch_types=[
            pltpu.VMEM((bpw,), jnp.int32),
            pltpu.VMEM((bpw, _DIM), jnp.float32),
            pltpu.SemaphoreType.DMA,
        ],
    )
    def k(table_hbm, idx_hbm, out_hbm, idx_v, rows_v, sem):
        wid = lax.axis_index("s") * ncores + lax.axis_index("c")
        base = wid * bpw
        pltpu.sync_copy(idx_hbm.at[pl.ds(base, bpw)], idx_v)
        pltpu.async_copy(table_hbm.at[idx_v], rows_v, sem).wait()
        pltpu.sync_copy(rows_v, out_hbm.at[pl.ds(base, bpw)])

    return k(table, idx)


def _hist_body(ind_ref, perp_ref, h_ref):
    b = pl.program_id(0)
    nb = pl.num_programs(0)
    code = lax.broadcasted_iota(jnp.int32, (1, _TN), 1) + b * _TN

    def tok_step(t, acc):
        sub = ind_ref[pl.ds(t * _TM, _TM), :]
        eq = (sub == code).astype(jnp.float32)
        return acc + jnp.sum(eq, axis=0, keepdims=True)

    counts = lax.fori_loop(0, _N_TOKENS // _TM, tok_step,
                           jnp.zeros((1, _TN), jnp.float32))
    p = counts * (1.0 / _N_TOKENS)
    h_part = jnp.sum(p * jnp.log(p + 1e-10))
    h_prev = jnp.where(b == 0, 0.0, h_ref[0])
    h_new = h_prev + h_part
    h_ref[0] = h_new

    @pl.when(b == nb - 1)
    def _():
        perp_ref[...] = jnp.full((1, 1), jnp.exp(-h_new), jnp.float32)


def _hist_call(ind2):
    return pl.pallas_call(
        _hist_body,
        grid=(_NBATCH,),
        in_specs=[pl.BlockSpec((_N_TOKENS, 1), lambda b: (0, 0))],
        out_specs=pl.BlockSpec((1, 1), lambda b: (0, 0)),
        out_shape=jax.ShapeDtypeStruct((1, 1), jnp.float32),
        scratch_shapes=[pltpu.SMEM((1,), jnp.float32)],
        compiler_params=pltpu.CompilerParams(
            dimension_semantics=("arbitrary",)),
    )(ind2)


def _diff_body(fl_ref, q_ref, diff_ref):
    b = pl.program_id(0)
    d = q_ref[...] - fl_ref[...]
    val = jnp.sum(d * d) * (1.0 / (_TB * _DIM))
    lane8 = lax.broadcasted_iota(jnp.int32, (1, _NBATCH), 1)
    diff_ref[...] = jnp.where(lane8 == b, val, diff_ref[...])


def _diff_call(flatten, quantize):
    return pl.pallas_call(
        _diff_body,
        grid=(_NBATCH,),
        in_specs=[
            pl.BlockSpec((_TB, _DIM), lambda b: (b, 0)),
            pl.BlockSpec((_TB, _DIM), lambda b: (b, 0)),
        ],
        out_specs=pl.BlockSpec((1, _NBATCH), lambda b: (0, 0)),
        out_shape=jax.ShapeDtypeStruct((1, _NBATCH), jnp.float32),
        compiler_params=pltpu.CompilerParams(
            dimension_semantics=("arbitrary",)),
    )(flatten, quantize)


def kernel(x, embed):
    xp = jnp.transpose(x, (0, 2, 3, 1))
    flatten = xp.reshape(-1, _DIM)
    e2 = (embed ** 2).sum(axis=0, keepdims=True)
    xb = (2.0 * flatten).astype(jnp.bfloat16)
    ind2 = _argmin_call(e2, xb, embed)
    perp11 = _hist_call(ind2)
    quantize = _sc_gather(embed.T, ind2.reshape(_N_TOKENS))
    diff_row = _diff_call(flatten, quantize)
    out0 = jnp.transpose(quantize.reshape(8, 32, 32, _DIM), (0, 3, 1, 2))
    return out0, diff_row.reshape(_NBATCH, 1), perp11.reshape(())
